# baseline (speedup 1.0000x reference)
import functools

import jax
import jax.numpy as jnp
from jax.experimental import pallas as pl

_EPS = 1e-08
_M = 256


def _stats1_kernel(x_ref, w_ref, b_ref, acc_ref):
    i = pl.program_id(0)
    y = jnp.dot(x_ref[...], w_ref[...].T, preferred_element_type=jnp.float32)
    y = y + b_ref[...]
    s = jnp.sum(y, axis=0, keepdims=True)
    ss = jnp.sum(y * y, axis=0, keepdims=True)
    blk = jnp.concatenate([s, ss], axis=0)

    @pl.when(i == 0)
    def _():
        acc_ref[...] = blk

    @pl.when(i > 0)
    def _():
        acc_ref[...] += blk


def _f2_kernel(x_ref, w_ref, b_ref, g_ref, beta_ref, out_ref):
    y = jnp.dot(x_ref[...], w_ref[...].T, preferred_element_type=jnp.float32)
    y = y + b_ref[...]
    m = jnp.mean(y, axis=0, keepdims=True)
    v = jnp.mean((y - m) ** 2, axis=0, keepdims=True)
    out_ref[...] = jnp.maximum(
        (y - m) / jnp.sqrt(v + 1e-5) * g_ref[...] + beta_ref[...], 0.0
    )


def _main_kernel(x1_ref, p1_ref, p2t_ref, f2_ref, w1_ref, b1_ref, g1_ref,
                 beta1_ref, stats_ref, out_ref, *, n_rows, n2):
    mean = stats_ref[0:1, :] / n_rows
    var = stats_ref[1:2, :] / n_rows - mean * mean
    x1 = x1_ref[0]
    y1 = jnp.dot(x1, w1_ref[...].T, preferred_element_type=jnp.float32)
    y1 = y1 + b1_ref[...]
    f1 = jnp.maximum(
        (y1 - mean) / jnp.sqrt(var + 1e-5) * g1_ref[...] + beta1_ref[...], 0.0
    )

    p1 = p1_ref[0]
    p2t = p2t_ref[0]
    q2 = jnp.sum(p1 * p1, axis=1, keepdims=True)
    r2 = jnp.sum(p2t * p2t, axis=0, keepdims=True)
    cross = jnp.dot(p1, p2t, preferred_element_type=jnp.float32)
    d2 = jnp.maximum(q2 + r2 - 2.0 * cross, 0.0)

    iota = jax.lax.broadcasted_iota(jnp.int32, d2.shape, 1)
    big = jnp.float32(3e38)
    d = d2
    mins = []
    sels = []
    for _ in range(3):
        mk = jnp.min(d, axis=1, keepdims=True)
        idxk = jnp.min(jnp.where(d == mk, iota, n2), axis=1, keepdims=True)
        sel = iota == idxk
        mins.append(mk)
        sels.append(sel)
        d = jnp.where(sel, big, d)
    r1 = 1.0 / (jnp.sqrt(mins[0]) + _EPS)
    r2w = 1.0 / (jnp.sqrt(mins[1]) + _EPS)
    r3 = 1.0 / (jnp.sqrt(mins[2]) + _EPS)
    norm = r1 + r2w + r3
    oh = jnp.where(
        sels[0], r1 / norm,
        jnp.where(sels[1], r2w / norm,
                  jnp.where(sels[2], r3 / norm, 0.0)),
    )
    new = jnp.dot(oh, f2_ref[0], preferred_element_type=jnp.float32)
    out_ref[0] = f1 + new


def kernel(feats1, points1, feats2, points2, W1, b1, g1, beta1, W2, b2, g2,
           beta2):
    B, N1, C1 = feats1.shape
    _, N2, C2 = feats2.shape
    C = W1.shape[0]
    x1 = feats1.reshape(B * N1, C1)
    x2 = feats2.reshape(B * N2, C2)
    b1r = b1.reshape(1, C)
    g1r = g1.reshape(1, C)
    beta1r = beta1.reshape(1, C)
    b2r = b2.reshape(1, C)
    g2r = g2.reshape(1, C)
    beta2r = beta2.reshape(1, C)

    chunk = 4096
    nchunks = (B * N1) // chunk
    stats = pl.pallas_call(
        _stats1_kernel,
        grid=(nchunks,),
        in_specs=[
            pl.BlockSpec((chunk, C1), lambda i: (i, 0)),
            pl.BlockSpec((C, C1), lambda i: (0, 0)),
            pl.BlockSpec((1, C), lambda i: (0, 0)),
        ],
        out_specs=pl.BlockSpec((2, C), lambda i: (0, 0)),
        out_shape=jax.ShapeDtypeStruct((2, C), jnp.float32),
    )(x1, W1, b1r)

    f2 = pl.pallas_call(
        _f2_kernel,
        out_shape=jax.ShapeDtypeStruct((B * N2, C), jnp.float32),
    )(x2, W2, b2r, g2r, beta2r)
    f2 = f2.reshape(B, N2, C)

    p2t = points2.transpose(0, 2, 1)

    nb = N1 // _M
    out = pl.pallas_call(
        functools.partial(_main_kernel, n_rows=float(B * N1), n2=N2),
        grid=(B, nb),
        in_specs=[
            pl.BlockSpec((1, _M, C1), lambda b, n: (b, n, 0)),
            pl.BlockSpec((1, _M, 3), lambda b, n: (b, n, 0)),
            pl.BlockSpec((1, 3, N2), lambda b, n: (b, 0, 0)),
            pl.BlockSpec((1, N2, C), lambda b, n: (b, 0, 0)),
            pl.BlockSpec((C, C1), lambda b, n: (0, 0)),
            pl.BlockSpec((1, C), lambda b, n: (0, 0)),
            pl.BlockSpec((1, C), lambda b, n: (0, 0)),
            pl.BlockSpec((1, C), lambda b, n: (0, 0)),
            pl.BlockSpec((2, C), lambda b, n: (0, 0)),
        ],
        out_specs=pl.BlockSpec((1, _M, C), lambda b, n: (b, n, 0)),
        out_shape=jax.ShapeDtypeStruct((B, N1, C), jnp.float32),
    )(feats1, points1, p2t, f2, W1, b1r, g1r, beta1r, stats)

    return (out, points1)


# speedup vs baseline: 1.1576x; 1.1576x over previous
import functools

import jax
import jax.numpy as jnp
from jax.experimental import pallas as pl

_EPS = 1e-08
_M = 256


def _stats1_kernel(x_ref, w_ref, b_ref, acc_ref):
    i = pl.program_id(0)
    y = jnp.dot(x_ref[...], w_ref[...].T, preferred_element_type=jnp.float32)
    y = y + b_ref[...]
    s = jnp.sum(y, axis=0, keepdims=True)
    ss = jnp.sum(y * y, axis=0, keepdims=True)
    blk = jnp.concatenate([s, ss], axis=0)

    @pl.when(i == 0)
    def _():
        acc_ref[...] = blk

    @pl.when(i > 0)
    def _():
        acc_ref[...] += blk


def _f2_kernel(x_ref, w_ref, b_ref, g_ref, beta_ref, out_ref):
    y = jnp.dot(x_ref[...], w_ref[...].T, preferred_element_type=jnp.float32)
    y = y + b_ref[...]
    m = jnp.mean(y, axis=0, keepdims=True)
    v = jnp.mean((y - m) ** 2, axis=0, keepdims=True)
    out_ref[...] = jnp.maximum(
        (y - m) / jnp.sqrt(v + 1e-5) * g_ref[...] + beta_ref[...], 0.0
    )


def _main_kernel(x1_ref, p1_ref, p2t_ref, f2_ref, w1_ref, b1_ref, g1_ref,
                 beta1_ref, stats_ref, out_ref, *, n_rows, n2):
    mean = stats_ref[0:1, :] / n_rows
    var = stats_ref[1:2, :] / n_rows - mean * mean
    x1 = x1_ref[0]
    y1 = jnp.dot(x1, w1_ref[...].T, preferred_element_type=jnp.float32)
    y1 = y1 + b1_ref[...]
    f1 = jnp.maximum(
        (y1 - mean) / jnp.sqrt(var + 1e-5) * g1_ref[...] + beta1_ref[...], 0.0
    )

    p1 = p1_ref[0]
    p2t = p2t_ref[0]
    q2 = jnp.sum(p1 * p1, axis=1, keepdims=True)
    r2 = jnp.sum(p2t * p2t, axis=0, keepdims=True)
    cross = jnp.dot(p1, p2t, preferred_element_type=jnp.float32)
    d2 = jnp.maximum(q2 + r2 - 2.0 * cross, 0.0)

    iota = jax.lax.broadcasted_iota(jnp.int32, d2.shape, 1)
    hi = jnp.int32(-2048)
    ukey = (jax.lax.bitcast_convert_type(d2, jnp.int32) & hi) | iota
    ibig = jnp.int32(0x7FFFFFFF)
    m1 = jnp.min(ukey, axis=1, keepdims=True)
    u2 = jnp.where(ukey == m1, ibig, ukey)
    m2 = jnp.min(u2, axis=1, keepdims=True)
    m3 = jnp.min(jnp.where(u2 == m2, ibig, u2), axis=1, keepdims=True)
    sels = [ukey == m1, ukey == m2, ukey == m3]
    mins = [jax.lax.bitcast_convert_type(mk & hi, jnp.float32) for mk in (m1, m2, m3)]
    r1 = 1.0 / (jnp.sqrt(mins[0]) + _EPS)
    r2w = 1.0 / (jnp.sqrt(mins[1]) + _EPS)
    r3 = 1.0 / (jnp.sqrt(mins[2]) + _EPS)
    norm = r1 + r2w + r3
    oh = jnp.where(
        sels[0], r1 / norm,
        jnp.where(sels[1], r2w / norm,
                  jnp.where(sels[2], r3 / norm, 0.0)),
    )
    new = jnp.dot(oh, f2_ref[0], preferred_element_type=jnp.float32)
    out_ref[0] = f1 + new


def kernel(feats1, points1, feats2, points2, W1, b1, g1, beta1, W2, b2, g2,
           beta2):
    B, N1, C1 = feats1.shape
    _, N2, C2 = feats2.shape
    C = W1.shape[0]
    x1 = feats1.reshape(B * N1, C1)
    x2 = feats2.reshape(B * N2, C2)
    b1r = b1.reshape(1, C)
    g1r = g1.reshape(1, C)
    beta1r = beta1.reshape(1, C)
    b2r = b2.reshape(1, C)
    g2r = g2.reshape(1, C)
    beta2r = beta2.reshape(1, C)

    chunk = 4096
    nchunks = (B * N1) // chunk
    stats = pl.pallas_call(
        _stats1_kernel,
        grid=(nchunks,),
        in_specs=[
            pl.BlockSpec((chunk, C1), lambda i: (i, 0)),
            pl.BlockSpec((C, C1), lambda i: (0, 0)),
            pl.BlockSpec((1, C), lambda i: (0, 0)),
        ],
        out_specs=pl.BlockSpec((2, C), lambda i: (0, 0)),
        out_shape=jax.ShapeDtypeStruct((2, C), jnp.float32),
    )(x1, W1, b1r)

    f2 = pl.pallas_call(
        _f2_kernel,
        out_shape=jax.ShapeDtypeStruct((B * N2, C), jnp.float32),
    )(x2, W2, b2r, g2r, beta2r)
    f2 = f2.reshape(B, N2, C)

    p2t = points2.transpose(0, 2, 1)

    nb = N1 // _M
    out = pl.pallas_call(
        functools.partial(_main_kernel, n_rows=float(B * N1), n2=N2),
        grid=(B, nb),
        in_specs=[
            pl.BlockSpec((1, _M, C1), lambda b, n: (b, n, 0)),
            pl.BlockSpec((1, _M, 3), lambda b, n: (b, n, 0)),
            pl.BlockSpec((1, 3, N2), lambda b, n: (b, 0, 0)),
            pl.BlockSpec((1, N2, C), lambda b, n: (b, 0, 0)),
            pl.BlockSpec((C, C1), lambda b, n: (0, 0)),
            pl.BlockSpec((1, C), lambda b, n: (0, 0)),
            pl.BlockSpec((1, C), lambda b, n: (0, 0)),
            pl.BlockSpec((1, C), lambda b, n: (0, 0)),
            pl.BlockSpec((2, C), lambda b, n: (0, 0)),
        ],
        out_specs=pl.BlockSpec((1, _M, C), lambda b, n: (b, n, 0)),
        out_shape=jax.ShapeDtypeStruct((B, N1, C), jnp.float32),
    )(feats1, points1, p2t, f2, W1, b1r, g1r, beta1r, stats)

    return (out, points1)


# speedup vs baseline: 1.3334x; 1.1518x over previous
import functools

import jax
import jax.numpy as jnp
from jax.experimental import pallas as pl

_EPS = 1e-08
_M = 256


def _stats1_kernel(x_ref, w_ref, b_ref, acc_ref):
    i = pl.program_id(0)
    y = jnp.dot(x_ref[...], w_ref[...].T, preferred_element_type=jnp.float32)
    y = y + b_ref[...]
    s = jnp.sum(y, axis=0, keepdims=True)
    ss = jnp.sum(y * y, axis=0, keepdims=True)
    blk = jnp.concatenate([s, ss], axis=0)

    @pl.when(i == 0)
    def _():
        acc_ref[...] = blk

    @pl.when(i > 0)
    def _():
        acc_ref[...] += blk


def _f2_kernel(x_ref, w_ref, b_ref, g_ref, beta_ref, out_ref):
    y = jnp.dot(x_ref[...], w_ref[...].T, preferred_element_type=jnp.float32)
    y = y + b_ref[...]
    m = jnp.mean(y, axis=0, keepdims=True)
    v = jnp.mean((y - m) ** 2, axis=0, keepdims=True)
    out_ref[...] = jnp.maximum(
        (y - m) / jnp.sqrt(v + 1e-5) * g_ref[...] + beta_ref[...], 0.0
    )


def _main_kernel(x1_ref, p1_ref, p2t_ref, f2_ref, w1_ref, b1_ref, g1_ref,
                 beta1_ref, stats_ref, out_ref, *, n_rows, n2):
    mean = stats_ref[0:1, :] / n_rows
    var = stats_ref[1:2, :] / n_rows - mean * mean
    x1 = x1_ref[0]
    y1 = jnp.dot(x1, w1_ref[...].T, preferred_element_type=jnp.float32)
    y1 = y1 + b1_ref[...]
    f1 = jnp.maximum(
        (y1 - mean) / jnp.sqrt(var + 1e-5) * g1_ref[...] + beta1_ref[...], 0.0
    )

    p1 = p1_ref[0]
    p2t = p2t_ref[0]
    q2 = jnp.sum(p1 * p1, axis=1, keepdims=True)
    r2 = jnp.sum(p2t * p2t, axis=0, keepdims=True)
    cross = jnp.dot(p1, p2t, preferred_element_type=jnp.float32)
    d2 = jnp.maximum(q2 + r2 - 2.0 * cross, 0.0)

    iota = jax.lax.broadcasted_iota(jnp.int32, d2.shape, 1)
    hi = jnp.int32(-2048)
    ukey = (jax.lax.bitcast_convert_type(d2, jnp.int32) & hi) | iota
    fkey = jax.lax.bitcast_convert_type(ukey + jnp.int32(0x00800000),
                                        jnp.float32)
    fbig = jnp.float32(3e38)
    m1 = jnp.min(fkey, axis=1, keepdims=True)
    u2 = jnp.where(fkey == m1, fbig, fkey)
    m2 = jnp.min(u2, axis=1, keepdims=True)
    m3 = jnp.min(jnp.where(u2 == m2, fbig, u2), axis=1, keepdims=True)
    sels = [fkey == m1, fkey == m2, fkey == m3]
    mins = [
        jax.lax.bitcast_convert_type(
            (jax.lax.bitcast_convert_type(mk, jnp.int32)
             - jnp.int32(0x00800000)) & hi,
            jnp.float32,
        )
        for mk in (m1, m2, m3)
    ]
    r1 = 1.0 / (jnp.sqrt(mins[0]) + _EPS)
    r2w = 1.0 / (jnp.sqrt(mins[1]) + _EPS)
    r3 = 1.0 / (jnp.sqrt(mins[2]) + _EPS)
    norm = r1 + r2w + r3
    oh = jnp.where(
        sels[0], r1 / norm,
        jnp.where(sels[1], r2w / norm,
                  jnp.where(sels[2], r3 / norm, 0.0)),
    )
    new = jnp.dot(oh, f2_ref[0], preferred_element_type=jnp.float32)
    out_ref[0] = f1 + new


def kernel(feats1, points1, feats2, points2, W1, b1, g1, beta1, W2, b2, g2,
           beta2):
    B, N1, C1 = feats1.shape
    _, N2, C2 = feats2.shape
    C = W1.shape[0]
    x1 = feats1.reshape(B * N1, C1)
    x2 = feats2.reshape(B * N2, C2)
    b1r = b1.reshape(1, C)
    g1r = g1.reshape(1, C)
    beta1r = beta1.reshape(1, C)
    b2r = b2.reshape(1, C)
    g2r = g2.reshape(1, C)
    beta2r = beta2.reshape(1, C)

    chunk = 4096
    nchunks = (B * N1) // chunk
    stats = pl.pallas_call(
        _stats1_kernel,
        grid=(nchunks,),
        in_specs=[
            pl.BlockSpec((chunk, C1), lambda i: (i, 0)),
            pl.BlockSpec((C, C1), lambda i: (0, 0)),
            pl.BlockSpec((1, C), lambda i: (0, 0)),
        ],
        out_specs=pl.BlockSpec((2, C), lambda i: (0, 0)),
        out_shape=jax.ShapeDtypeStruct((2, C), jnp.float32),
    )(x1, W1, b1r)

    f2 = pl.pallas_call(
        _f2_kernel,
        out_shape=jax.ShapeDtypeStruct((B * N2, C), jnp.float32),
    )(x2, W2, b2r, g2r, beta2r)
    f2 = f2.reshape(B, N2, C)

    p2t = points2.transpose(0, 2, 1)

    nb = N1 // _M
    out = pl.pallas_call(
        functools.partial(_main_kernel, n_rows=float(B * N1), n2=N2),
        grid=(B, nb),
        in_specs=[
            pl.BlockSpec((1, _M, C1), lambda b, n: (b, n, 0)),
            pl.BlockSpec((1, _M, 3), lambda b, n: (b, n, 0)),
            pl.BlockSpec((1, 3, N2), lambda b, n: (b, 0, 0)),
            pl.BlockSpec((1, N2, C), lambda b, n: (b, 0, 0)),
            pl.BlockSpec((C, C1), lambda b, n: (0, 0)),
            pl.BlockSpec((1, C), lambda b, n: (0, 0)),
            pl.BlockSpec((1, C), lambda b, n: (0, 0)),
            pl.BlockSpec((1, C), lambda b, n: (0, 0)),
            pl.BlockSpec((2, C), lambda b, n: (0, 0)),
        ],
        out_specs=pl.BlockSpec((1, _M, C), lambda b, n: (b, n, 0)),
        out_shape=jax.ShapeDtypeStruct((B, N1, C), jnp.float32),
    )(feats1, points1, p2t, f2, W1, b1r, g1r, beta1r, stats)

    return (out, points1)


# speedup vs baseline: 1.4253x; 1.0689x over previous
import functools

import jax
import jax.numpy as jnp
from jax.experimental import pallas as pl

_EPS = 1e-08
_M = 512


def _stats1_kernel(x_ref, w_ref, b_ref, acc_ref):
    i = pl.program_id(0)
    y = jnp.dot(x_ref[...], w_ref[...].T, preferred_element_type=jnp.float32)
    y = y + b_ref[...]
    s = jnp.sum(y, axis=0, keepdims=True)
    ss = jnp.sum(y * y, axis=0, keepdims=True)
    blk = jnp.concatenate([s, ss], axis=0)

    @pl.when(i == 0)
    def _():
        acc_ref[...] = blk

    @pl.when(i > 0)
    def _():
        acc_ref[...] += blk


def _f2_kernel(x_ref, w_ref, b_ref, g_ref, beta_ref, out_ref):
    y = jnp.dot(x_ref[...], w_ref[...].T, preferred_element_type=jnp.float32)
    y = y + b_ref[...]
    m = jnp.mean(y, axis=0, keepdims=True)
    v = jnp.mean((y - m) ** 2, axis=0, keepdims=True)
    out_ref[...] = jnp.maximum(
        (y - m) / jnp.sqrt(v + 1e-5) * g_ref[...] + beta_ref[...], 0.0
    )


def _main_kernel(x1_ref, p1_ref, p2t_ref, f2_ref, w1_ref, b1_ref, g1_ref,
                 beta1_ref, stats_ref, out_ref, *, n_rows, n2):
    mean = stats_ref[0:1, :] / n_rows
    var = stats_ref[1:2, :] / n_rows - mean * mean
    x1 = x1_ref[0]
    y1 = jnp.dot(x1, w1_ref[...].T, preferred_element_type=jnp.float32)
    y1 = y1 + b1_ref[...]
    f1 = jnp.maximum(
        (y1 - mean) / jnp.sqrt(var + 1e-5) * g1_ref[...] + beta1_ref[...], 0.0
    )

    p1 = p1_ref[0]
    p2t = p2t_ref[0]
    q2 = jnp.sum(p1 * p1, axis=1, keepdims=True)
    r2 = jnp.sum(p2t * p2t, axis=0, keepdims=True)
    cross = jnp.dot(p1, p2t, preferred_element_type=jnp.float32)
    d2 = jnp.maximum(q2 + r2 - 2.0 * cross, 0.0)

    iota = jax.lax.broadcasted_iota(jnp.int32, d2.shape, 1)
    hi = jnp.int32(-2048)
    ukey = (jax.lax.bitcast_convert_type(d2, jnp.int32) & hi) | iota
    fkey = jax.lax.bitcast_convert_type(ukey + jnp.int32(0x00800000),
                                        jnp.float32)
    fbig = jnp.float32(3e38)
    m1 = jnp.min(fkey, axis=1, keepdims=True)
    u2 = jnp.where(fkey == m1, fbig, fkey)
    m2 = jnp.min(u2, axis=1, keepdims=True)
    m3 = jnp.min(jnp.where(u2 == m2, fbig, u2), axis=1, keepdims=True)
    sels = [fkey == m1, fkey == m2, fkey == m3]
    mins = [
        jax.lax.bitcast_convert_type(
            (jax.lax.bitcast_convert_type(mk, jnp.int32)
             - jnp.int32(0x00800000)) & hi,
            jnp.float32,
        )
        for mk in (m1, m2, m3)
    ]
    r1 = 1.0 / (jnp.sqrt(mins[0]) + _EPS)
    r2w = 1.0 / (jnp.sqrt(mins[1]) + _EPS)
    r3 = 1.0 / (jnp.sqrt(mins[2]) + _EPS)
    norm = r1 + r2w + r3
    oh = jnp.where(
        sels[0], r1 / norm,
        jnp.where(sels[1], r2w / norm,
                  jnp.where(sels[2], r3 / norm, 0.0)),
    )
    new = jnp.dot(oh, f2_ref[0], preferred_element_type=jnp.float32)
    out_ref[0] = f1 + new


def kernel(feats1, points1, feats2, points2, W1, b1, g1, beta1, W2, b2, g2,
           beta2):
    B, N1, C1 = feats1.shape
    _, N2, C2 = feats2.shape
    C = W1.shape[0]
    x1 = feats1.reshape(B * N1, C1)
    x2 = feats2.reshape(B * N2, C2)
    b1r = b1.reshape(1, C)
    g1r = g1.reshape(1, C)
    beta1r = beta1.reshape(1, C)
    b2r = b2.reshape(1, C)
    g2r = g2.reshape(1, C)
    beta2r = beta2.reshape(1, C)

    chunk = 4096
    nchunks = (B * N1) // chunk
    stats = pl.pallas_call(
        _stats1_kernel,
        grid=(nchunks,),
        in_specs=[
            pl.BlockSpec((chunk, C1), lambda i: (i, 0)),
            pl.BlockSpec((C, C1), lambda i: (0, 0)),
            pl.BlockSpec((1, C), lambda i: (0, 0)),
        ],
        out_specs=pl.BlockSpec((2, C), lambda i: (0, 0)),
        out_shape=jax.ShapeDtypeStruct((2, C), jnp.float32),
    )(x1, W1, b1r)

    f2 = pl.pallas_call(
        _f2_kernel,
        out_shape=jax.ShapeDtypeStruct((B * N2, C), jnp.float32),
    )(x2, W2, b2r, g2r, beta2r)
    f2 = f2.reshape(B, N2, C)

    p2t = points2.transpose(0, 2, 1)

    nb = N1 // _M
    out = pl.pallas_call(
        functools.partial(_main_kernel, n_rows=float(B * N1), n2=N2),
        grid=(B, nb),
        in_specs=[
            pl.BlockSpec((1, _M, C1), lambda b, n: (b, n, 0)),
            pl.BlockSpec((1, _M, 3), lambda b, n: (b, n, 0)),
            pl.BlockSpec((1, 3, N2), lambda b, n: (b, 0, 0)),
            pl.BlockSpec((1, N2, C), lambda b, n: (b, 0, 0)),
            pl.BlockSpec((C, C1), lambda b, n: (0, 0)),
            pl.BlockSpec((1, C), lambda b, n: (0, 0)),
            pl.BlockSpec((1, C), lambda b, n: (0, 0)),
            pl.BlockSpec((1, C), lambda b, n: (0, 0)),
            pl.BlockSpec((2, C), lambda b, n: (0, 0)),
        ],
        out_specs=pl.BlockSpec((1, _M, C), lambda b, n: (b, n, 0)),
        out_shape=jax.ShapeDtypeStruct((B, N1, C), jnp.float32),
    )(feats1, points1, p2t, f2, W1, b1r, g1r, beta1r, stats)

    return (out, points1)


# speedup vs baseline: 1.4781x; 1.0371x over previous
import functools

import jax
import jax.numpy as jnp
from jax.experimental import pallas as pl

_EPS = 1e-08
_M = 1024


def _stats1_kernel(x_ref, w_ref, b_ref, acc_ref):
    i = pl.program_id(0)
    y = jnp.dot(x_ref[...], w_ref[...].T, preferred_element_type=jnp.float32)
    y = y + b_ref[...]
    s = jnp.sum(y, axis=0, keepdims=True)
    ss = jnp.sum(y * y, axis=0, keepdims=True)
    blk = jnp.concatenate([s, ss], axis=0)

    @pl.when(i == 0)
    def _():
        acc_ref[...] = blk

    @pl.when(i > 0)
    def _():
        acc_ref[...] += blk


def _f2_kernel(x_ref, w_ref, b_ref, g_ref, beta_ref, out_ref):
    y = jnp.dot(x_ref[...], w_ref[...].T, preferred_element_type=jnp.float32)
    y = y + b_ref[...]
    m = jnp.mean(y, axis=0, keepdims=True)
    v = jnp.mean((y - m) ** 2, axis=0, keepdims=True)
    out_ref[...] = jnp.maximum(
        (y - m) / jnp.sqrt(v + 1e-5) * g_ref[...] + beta_ref[...], 0.0
    )


def _main_kernel(x1_ref, p1_ref, p2t_ref, f2_ref, w1_ref, b1_ref, g1_ref,
                 beta1_ref, stats_ref, out_ref, *, n_rows, n2):
    mean = stats_ref[0:1, :] / n_rows
    var = stats_ref[1:2, :] / n_rows - mean * mean
    x1 = x1_ref[0]
    y1 = jnp.dot(x1, w1_ref[...].T, preferred_element_type=jnp.float32)
    y1 = y1 + b1_ref[...]
    f1 = jnp.maximum(
        (y1 - mean) / jnp.sqrt(var + 1e-5) * g1_ref[...] + beta1_ref[...], 0.0
    )

    p1 = p1_ref[0]
    p2t = p2t_ref[0]
    q2 = jnp.sum(p1 * p1, axis=1, keepdims=True)
    r2 = jnp.sum(p2t * p2t, axis=0, keepdims=True)
    cross = jnp.dot(p1, p2t, preferred_element_type=jnp.float32)
    d2 = jnp.maximum(q2 + r2 - 2.0 * cross, 0.0)

    iota = jax.lax.broadcasted_iota(jnp.int32, d2.shape, 1)
    hi = jnp.int32(-2048)
    ukey = (jax.lax.bitcast_convert_type(d2, jnp.int32) & hi) | iota
    fkey = jax.lax.bitcast_convert_type(ukey + jnp.int32(0x00800000),
                                        jnp.float32)
    fbig = jnp.float32(3e38)
    m1 = jnp.min(fkey, axis=1, keepdims=True)
    u2 = jnp.where(fkey == m1, fbig, fkey)
    m2 = jnp.min(u2, axis=1, keepdims=True)
    m3 = jnp.min(jnp.where(u2 == m2, fbig, u2), axis=1, keepdims=True)
    sels = [fkey == m1, fkey == m2, fkey == m3]
    mins = [
        jax.lax.bitcast_convert_type(
            (jax.lax.bitcast_convert_type(mk, jnp.int32)
             - jnp.int32(0x00800000)) & hi,
            jnp.float32,
        )
        for mk in (m1, m2, m3)
    ]
    r1 = 1.0 / (jnp.sqrt(mins[0]) + _EPS)
    r2w = 1.0 / (jnp.sqrt(mins[1]) + _EPS)
    r3 = 1.0 / (jnp.sqrt(mins[2]) + _EPS)
    norm = r1 + r2w + r3
    oh = jnp.where(
        sels[0], r1 / norm,
        jnp.where(sels[1], r2w / norm,
                  jnp.where(sels[2], r3 / norm, 0.0)),
    )
    new = jnp.dot(oh, f2_ref[0], preferred_element_type=jnp.float32)
    out_ref[0] = f1 + new


def kernel(feats1, points1, feats2, points2, W1, b1, g1, beta1, W2, b2, g2,
           beta2):
    B, N1, C1 = feats1.shape
    _, N2, C2 = feats2.shape
    C = W1.shape[0]
    x1 = feats1.reshape(B * N1, C1)
    x2 = feats2.reshape(B * N2, C2)
    b1r = b1.reshape(1, C)
    g1r = g1.reshape(1, C)
    beta1r = beta1.reshape(1, C)
    b2r = b2.reshape(1, C)
    g2r = g2.reshape(1, C)
    beta2r = beta2.reshape(1, C)

    chunk = 4096
    nchunks = (B * N1) // chunk
    stats = pl.pallas_call(
        _stats1_kernel,
        grid=(nchunks,),
        in_specs=[
            pl.BlockSpec((chunk, C1), lambda i: (i, 0)),
            pl.BlockSpec((C, C1), lambda i: (0, 0)),
            pl.BlockSpec((1, C), lambda i: (0, 0)),
        ],
        out_specs=pl.BlockSpec((2, C), lambda i: (0, 0)),
        out_shape=jax.ShapeDtypeStruct((2, C), jnp.float32),
    )(x1, W1, b1r)

    f2 = pl.pallas_call(
        _f2_kernel,
        out_shape=jax.ShapeDtypeStruct((B * N2, C), jnp.float32),
    )(x2, W2, b2r, g2r, beta2r)
    f2 = f2.reshape(B, N2, C)

    p2t = points2.transpose(0, 2, 1)

    nb = N1 // _M
    out = pl.pallas_call(
        functools.partial(_main_kernel, n_rows=float(B * N1), n2=N2),
        grid=(B, nb),
        in_specs=[
            pl.BlockSpec((1, _M, C1), lambda b, n: (b, n, 0)),
            pl.BlockSpec((1, _M, 3), lambda b, n: (b, n, 0)),
            pl.BlockSpec((1, 3, N2), lambda b, n: (b, 0, 0)),
            pl.BlockSpec((1, N2, C), lambda b, n: (b, 0, 0)),
            pl.BlockSpec((C, C1), lambda b, n: (0, 0)),
            pl.BlockSpec((1, C), lambda b, n: (0, 0)),
            pl.BlockSpec((1, C), lambda b, n: (0, 0)),
            pl.BlockSpec((1, C), lambda b, n: (0, 0)),
            pl.BlockSpec((2, C), lambda b, n: (0, 0)),
        ],
        out_specs=pl.BlockSpec((1, _M, C), lambda b, n: (b, n, 0)),
        out_shape=jax.ShapeDtypeStruct((B, N1, C), jnp.float32),
    )(feats1, points1, p2t, f2, W1, b1r, g1r, beta1r, stats)

    return (out, points1)
